# Initial kernel scaffold; baseline (speedup 1.0000x reference)
import jax
import jax.numpy as jnp
from jax.experimental import pallas as pl


def kernel(edge_index, edge_weight, ctx_ids, miss_ids, vocab_to_fg, embedding, fg_embedding, gc_W0, gc_W1, bn0_gamma, bn0_beta, bn1_gamma, bn1_beta, ctx_attn_W, ctx_attn_b, fusion_W, fusion_b, proj0_W, proj0_b, proj1_W, proj1_b):
    raise NotImplementedError("write your pallas kernel here")


# speedup vs baseline: 3.4217x; 3.4217x over previous
import functools
import math

import jax
import jax.numpy as jnp
from jax import lax
from jax.experimental import pallas as pl
from jax.experimental.pallas import tpu as pltpu
from jax.experimental.pallas import tpu_sc as plsc

NC = 2
NS = 16
LANES = 16
NW = NC * NS
BN_INV = 1.0 / math.sqrt(1.0 + 1e-5)


def _spmm_sc(N, Dm, E):
  epw = E // NW
  C = 80
  n_chunks = epw // C
  ZC = 80
  n_zchunks = N // ZC
  n_zrounds = (n_zchunks + NS - 1) // NS
  nq = Dm // LANES

  @functools.partial(
      pl.kernel,
      out_type=jax.ShapeDtypeStruct((NC, N, Dm), jnp.float32),
      mesh=plsc.VectorSubcoreMesh(core_axis_name="c", subcore_axis_name="s"),
      scratch_types=[
          pltpu.VMEM((C,), jnp.int32),
          pltpu.VMEM((C,), jnp.int32),
          pltpu.VMEM((C,), jnp.float32),
          pltpu.VMEM((C, Dm), jnp.float32),
          pltpu.VMEM((ZC, Dm), jnp.float32),
          pltpu.VMEM_SHARED((N, Dm), jnp.float32),
          pltpu.SemaphoreType.DMA,
      ],
  )
  def spmm(h_hbm, dst_hbm, src_hbm, w_hbm, out_hbm,
           idxd_v, idxs_v, w_v, rows_v, stage_v, acc_s, sem):
    cid = lax.axis_index("c")
    sid = lax.axis_index("s")
    wid = cid * NS + sid

    zv = jnp.zeros((LANES,), jnp.float32)

    def zrow(r, carry):
      for q in range(nq):
        stage_v[r, pl.ds(q * LANES, LANES)] = zv
      return carry

    lax.fori_loop(0, ZC, zrow, 0)

    def zcp(j, carry):
      c = j * NS + sid

      @pl.when(c < n_zchunks)
      def _():
        pltpu.sync_copy(stage_v, acc_s.at[pl.ds(c * ZC, ZC)])

      return carry

    lax.fori_loop(0, n_zrounds, zcp, 0)
    plsc.subcore_barrier()

    base0 = wid * epw

    def chunk(k, carry):
      base = base0 + k * C
      pltpu.sync_copy(dst_hbm.at[pl.ds(base, C)], idxd_v)
      pltpu.sync_copy(src_hbm.at[pl.ds(base, C)], idxs_v)
      pltpu.sync_copy(w_hbm.at[pl.ds(base, C)], w_v)
      pltpu.async_copy(h_hbm.at[idxd_v], rows_v, sem).wait()

      def grp(g, c2):
        w16 = w_v[pl.ds(g * LANES, LANES)]
        for j in range(LANES):
          e = g * LANES + j
          wv = jnp.full((LANES,), w16[j], jnp.float32)
          for q in range(nq):
            sl = pl.ds(q * LANES, LANES)
            rows_v[e, sl] = rows_v[e, sl] * wv
        return c2

      lax.fori_loop(0, C // LANES, grp, 0)
      pltpu.sync_copy(rows_v, acc_s.at[idxs_v], add=True)
      return carry

    lax.fori_loop(0, n_chunks, chunk, 0)
    plsc.subcore_barrier()

    def ocp(j, carry):
      c = j * NS + sid

      @pl.when(c < n_zchunks)
      def _():
        pltpu.sync_copy(acc_s.at[pl.ds(c * ZC, ZC)], stage_v)
        pltpu.sync_copy(stage_v, out_hbm.at[cid, pl.ds(c * ZC, ZC)])

      return carry

    lax.fori_loop(0, n_zrounds, ocp, 0)

  return spmm


def _gathers_sc(V, N, Dm, B, L):
  T = B * L
  tpw = T // NW
  C = 80
  nch = tpw // C
  mpw = B // NW

  @functools.partial(
      pl.kernel,
      out_type=(
          jax.ShapeDtypeStruct((T, Dm), jnp.float32),
          jax.ShapeDtypeStruct((B, Dm), jnp.float32),
          jax.ShapeDtypeStruct((B,), jnp.int32),
          jax.ShapeDtypeStruct((B,), jnp.float32),
      ),
      mesh=plsc.VectorSubcoreMesh(core_axis_name="c", subcore_axis_name="s"),
      scratch_types=[
          pltpu.VMEM((C,), jnp.int32),
          pltpu.VMEM((C, Dm), jnp.float32),
          pltpu.VMEM((mpw,), jnp.int32),
          pltpu.VMEM((mpw, Dm), jnp.float32),
          pltpu.VMEM((mpw,), jnp.int32),
          pltpu.VMEM((mpw,), jnp.float32),
          pltpu.SemaphoreType.DMA,
      ],
  )
  def g1(emb_hbm, v2f_hbm, ctx_hbm, miss_hbm,
         ctx_out, base_out, fg_out, mask_out,
         cidx_v, crows_v, midx_v, mrows_v, fg_v, mk_v, sem):
    cid = lax.axis_index("c")
    sid = lax.axis_index("s")
    wid = cid * NS + sid

    def chunk(k, carry):
      base = wid * tpw + k * C
      pltpu.sync_copy(ctx_hbm.at[pl.ds(base, C)], cidx_v)
      pltpu.async_copy(emb_hbm.at[cidx_v], crows_v, sem).wait()
      pltpu.sync_copy(crows_v, ctx_out.at[pl.ds(base, C)])
      return carry

    lax.fori_loop(0, nch, chunk, 0)

    mb = wid * mpw
    pltpu.sync_copy(miss_hbm.at[pl.ds(mb, mpw)], midx_v)
    pltpu.async_copy(emb_hbm.at[midx_v], mrows_v, sem).wait()
    pltpu.sync_copy(mrows_v, base_out.at[pl.ds(mb, mpw)])

    pltpu.async_copy(v2f_hbm.at[midx_v], fg_v, sem).wait()
    for g in range(mpw // LANES):
      fg = fg_v[pl.ds(g * LANES, LANES)]
      mk = jnp.where(fg >= 0, 1.0, 0.0).astype(jnp.float32)
      fgc = jnp.clip(fg, 0, N - 1)
      fg_v[pl.ds(g * LANES, LANES)] = fgc
      mk_v[pl.ds(g * LANES, LANES)] = mk
    pltpu.sync_copy(fg_v, fg_out.at[pl.ds(mb, mpw)])
    pltpu.sync_copy(mk_v, mask_out.at[pl.ds(mb, mpw)])

  return g1


def _gather_graph_sc(N, Dm, B):
  mpw = B // NW
  nq = Dm // LANES

  @functools.partial(
      pl.kernel,
      out_type=jax.ShapeDtypeStruct((B, Dm), jnp.float32),
      mesh=plsc.VectorSubcoreMesh(core_axis_name="c", subcore_axis_name="s"),
      scratch_types=[
          pltpu.VMEM((mpw,), jnp.int32),
          pltpu.VMEM((mpw,), jnp.float32),
          pltpu.VMEM((mpw, Dm), jnp.float32),
          pltpu.SemaphoreType.DMA,
      ],
  )
  def g2(ge_hbm, fg_hbm, mk_hbm, out_hbm, idx_v, mk_v, rows_v, sem):
    cid = lax.axis_index("c")
    sid = lax.axis_index("s")
    wid = cid * NS + sid
    mb = wid * mpw
    pltpu.sync_copy(fg_hbm.at[pl.ds(mb, mpw)], idx_v)
    pltpu.sync_copy(mk_hbm.at[pl.ds(mb, mpw)], mk_v)
    pltpu.async_copy(ge_hbm.at[idx_v], rows_v, sem).wait()

    def row(g, carry):
      mk16 = mk_v[pl.ds(g * LANES, LANES)]
      for j in range(LANES):
        e = g * LANES + j
        mv = jnp.full((LANES,), mk16[j], jnp.float32)
        for q in range(nq):
          sl = pl.ds(q * LANES, LANES)
          rows_v[e, sl] = rows_v[e, sl] * mv
      return carry

    lax.fori_loop(0, mpw // LANES, row, 0)
    pltpu.sync_copy(rows_v, out_hbm.at[pl.ds(mb, mpw)])

  return g2


def _layer_tc(N, Dm, bm):
  def body(p_ref, h_ref, W_ref, g_ref, b_ref, o_ref):
    s = p_ref[0] + p_ref[1]
    y = lax.dot_general(s, W_ref[...], (((1,), (1,)), ((), ())),
                        preferred_element_type=jnp.float32)
    y = jnp.maximum(y, 0.0)
    y = y * (g_ref[...] * BN_INV) + b_ref[...]
    o_ref[...] = h_ref[...] + y

  return pl.pallas_call(
      body,
      grid=(N // bm,),
      in_specs=[
          pl.BlockSpec((NC, bm, Dm), lambda i: (0, i, 0)),
          pl.BlockSpec((bm, Dm), lambda i: (i, 0)),
          pl.BlockSpec((Dm, Dm), lambda i: (0, 0)),
          pl.BlockSpec((1, Dm), lambda i: (0, 0)),
          pl.BlockSpec((1, Dm), lambda i: (0, 0)),
      ],
      out_specs=pl.BlockSpec((bm, Dm), lambda i: (i, 0)),
      out_shape=jax.ShapeDtypeStruct((N, Dm), jnp.float32),
  )


def _attn_tc(B, L, Dm, bb):
  def body(c_ref, w_ref, o_ref):
    c = c_ref[...]
    lg = lax.dot_general(c, w_ref[...], (((2,), (1,)), ((), ())),
                         preferred_element_type=jnp.float32)[:, :, 0]
    m = jnp.max(lg, axis=1, keepdims=True)
    ex = jnp.exp(lg - m)
    a = ex / jnp.sum(ex, axis=1, keepdims=True)
    o_ref[...] = lax.dot_general(a, c, (((1,), (1,)), ((0,), (0,))),
                                 preferred_element_type=jnp.float32)

  return pl.pallas_call(
      body,
      grid=(B // bb,),
      in_specs=[
          pl.BlockSpec((bb, L, Dm), lambda i: (i, 0, 0)),
          pl.BlockSpec((1, Dm), lambda i: (0, 0)),
      ],
      out_specs=pl.BlockSpec((bb, Dm), lambda i: (i, 0)),
      out_shape=jax.ShapeDtypeStruct((B, Dm), jnp.float32),
  )


def _final_tc(B, Dm):
  def body(ctx_ref, base_ref, gp_ref, fW_ref, fb_ref,
           p0W_ref, p0b_ref, p1W_ref, p1b_ref, o_ref):
    x = jnp.concatenate([base_ref[...], gp_ref[...]], axis=1)
    miss = lax.dot_general(x, fW_ref[...], (((1,), (1,)), ((), ())),
                           preferred_element_type=jnp.float32) + fb_ref[...]
    q = jnp.concatenate([ctx_ref[...], miss], axis=1)
    hq = lax.dot_general(q, p0W_ref[...], (((1,), (1,)), ((), ())),
                         preferred_element_type=jnp.float32) + p0b_ref[...]
    hq = jnp.maximum(hq, 0.0)
    o_ref[...] = lax.dot_general(hq, p1W_ref[...], (((1,), (1,)), ((), ())),
                                 preferred_element_type=jnp.float32) + p1b_ref[...]

  return pl.pallas_call(
      body,
      out_shape=jax.ShapeDtypeStruct((B, Dm), jnp.float32),
  )


def kernel(edge_index, edge_weight, ctx_ids, miss_ids, vocab_to_fg, embedding,
           fg_embedding, gc_W0, gc_W1, bn0_gamma, bn0_beta, bn1_gamma,
           bn1_beta, ctx_attn_W, ctx_attn_b, fusion_W, fusion_b, proj0_W,
           proj0_b, proj1_W, proj1_b):
  N, Dm = fg_embedding.shape
  E = edge_weight.shape[0]
  B, L = ctx_ids.shape
  V = embedding.shape[0]

  src = edge_index[0]
  dst = edge_index[1]

  spmm = _spmm_sc(N, Dm, E)
  layer = _layer_tc(N, Dm, 1000)

  g0 = bn0_gamma.reshape(1, Dm)
  b0 = bn0_beta.reshape(1, Dm)
  g1w = bn1_gamma.reshape(1, Dm)
  b1w = bn1_beta.reshape(1, Dm)

  h1 = layer(spmm(fg_embedding, dst, src, edge_weight),
             fg_embedding, gc_W0, g0, b0)
  graph_embs = layer(spmm(h1, dst, src, edge_weight), h1, gc_W1, g1w, b1w)

  ctx_rows, base_emb, fg_clip, maskf = _gathers_sc(V, N, Dm, B, L)(
      embedding, vocab_to_fg, ctx_ids.reshape(B * L), miss_ids)
  ctx_emb = _attn_tc(B, L, Dm, 128)(ctx_rows.reshape(B, L, Dm), ctx_attn_W)
  gpart = _gather_graph_sc(N, Dm, B)(graph_embs, fg_clip, maskf)

  query = _final_tc(B, Dm)(
      ctx_emb, base_emb, gpart,
      fusion_W, fusion_b.reshape(1, Dm),
      proj0_W, proj0_b.reshape(1, 2 * Dm),
      proj1_W, proj1_b.reshape(1, Dm))

  return (query, graph_embs)


# speedup vs baseline: 6.8419x; 1.9996x over previous
import functools
import math

import jax
import jax.numpy as jnp
from jax import lax
from jax.experimental import pallas as pl
from jax.experimental.pallas import tpu as pltpu
from jax.experimental.pallas import tpu_sc as plsc

NC = 2
NS = 16
LANES = 16
NW = NC * NS
BN_INV = 1.0 / math.sqrt(1.0 + 1e-5)


def _spmm_sc(N, Dm, E):
  C = 40
  n_chunks = E // (NW * C)
  NB = 4
  NI = 8
  ZC = 200
  n_oc = N // ZC
  ZCz = C
  n_zc = N // ZCz
  nq = Dm // LANES

  @functools.partial(
      pl.kernel,
      out_type=jax.ShapeDtypeStruct((NC, N, Dm), jnp.float32),
      mesh=plsc.VectorSubcoreMesh(core_axis_name="c", subcore_axis_name="s"),
      scratch_types=[
          pltpu.VMEM((NI, 1, C), jnp.int32),
          pltpu.VMEM((NI, 1, C), jnp.int32),
          pltpu.VMEM((NI, 1, C), jnp.float32),
          pltpu.VMEM((NB, C, Dm), jnp.float32),
          pltpu.VMEM_SHARED((N, Dm), jnp.float32),
          [pltpu.SemaphoreType.DMA] * NB,
          [pltpu.SemaphoreType.DMA] * NB,
          [pltpu.SemaphoreType.DMA] * NI,
      ],
  )
  def spmm(h_hbm, dst_hbm, src_hbm, w_hbm, out_hbm,
           dstb, srcb, wb, rows_v, acc_s, gsems, ssems, isems):
    cid = lax.axis_index("c")
    sid = lax.axis_index("s")
    wid = cid * NS + sid

    zv = jnp.zeros((LANES,), jnp.float32)

    def zrow(r, carry):
      for q in range(nq):
        rows_v[0, r, pl.ds(q * LANES, LANES)] = zv
      return carry

    lax.fori_loop(0, C, zrow, 0)

    def zcp(j, carry):
      c = j * NS + sid

      @pl.when(c < n_zc)
      def _():
        pltpu.sync_copy(rows_v.at[0], acc_s.at[pl.ds(c * ZCz, ZCz)])

      return carry

    lax.fori_loop(0, (n_zc + NS - 1) // NS, zcp, 0)
    plsc.subcore_barrier()

    def issue_idx(k, i):
      pltpu.async_copy(dst_hbm.at[wid, k], dstb.at[i], isems[i])
      pltpu.async_copy(src_hbm.at[wid, k], srcb.at[i], isems[i])
      pltpu.async_copy(w_hbm.at[wid, k], wb.at[i], isems[i])

    def wait_idx(i):
      pltpu.make_async_copy(dst_hbm.at[wid, 0], dstb.at[i], isems[i]).wait()
      pltpu.make_async_copy(src_hbm.at[wid, 0], srcb.at[i], isems[i]).wait()
      pltpu.make_async_copy(w_hbm.at[wid, 0], wb.at[i], isems[i]).wait()

    def issue_gather(i, b):
      pltpu.async_copy(h_hbm.at[dstb.at[i, 0]], rows_v.at[b], gsems[b])

    def wait_gather(b):
      pltpu.make_async_copy(h_hbm.at[dstb.at[0, 0]], rows_v.at[b],
                            gsems[b]).wait()

    def issue_scatter(i, b):
      pltpu.async_copy(rows_v.at[b], acc_s.at[srcb.at[i, 0]], ssems[b],
                       add=True)

    def wait_scatter(b):
      pltpu.make_async_copy(rows_v.at[b], acc_s.at[srcb.at[0, 0]],
                            ssems[b]).wait()

    def scale(i, b):
      def grp(g, c2):
        w16 = wb[i, 0, pl.ds(g * LANES, LANES)]
        for j in range(LANES):
          e = g * LANES + j
          wv = jnp.full((LANES,), w16[j], jnp.float32)
          for q in range(nq):
            sl = pl.ds(q * LANES, LANES)
            rows_v[b, e, sl] = rows_v[b, e, sl] * wv
        return c2

      lax.fori_loop(0, C // LANES, grp, 0)
      if C % LANES:
        off = C - LANES
        w16 = wb[i, 0, pl.ds(off, LANES)]
        for j in range(LANES - (C % LANES), LANES):
          e = off + j
          wv = jnp.full((LANES,), w16[j], jnp.float32)
          for q in range(nq):
            sl = pl.ds(q * LANES, LANES)
            rows_v[b, e, sl] = rows_v[b, e, sl] * wv

    def step(k, u):
      k = jnp.int32(k)
      b = u % NB
      b2 = (u + 2) % NB
      i2 = (u + 2) % NI
      i4 = (u + 4) % NI

      @pl.when(k >= 2)
      def _():
        wait_scatter(b2)

      @pl.when(k + 2 < n_chunks)
      def _():
        wait_idx(i2)
        issue_gather(i2, b2)

      @pl.when(k + 4 < n_chunks)
      def _():
        issue_idx(k + 4, i4)

      wait_gather(b)
      scale(u % NI, b)
      issue_scatter(u % NI, b)

    for j in range(4):
      issue_idx(j, j)
    wait_idx(0)
    issue_gather(0, 0)
    wait_idx(1)
    issue_gather(1, 1)

    n_main = (n_chunks // NI) * NI

    def outer(g, carry):
      for u in range(NI):
        step(g * NI + u, u)
      return carry

    lax.fori_loop(0, n_main // NI, outer, 0)

    for k in range(n_main, n_chunks):
      step(k, k % NI)

    for k in range(n_chunks - 2, n_chunks):
      wait_scatter(k % NB)
    plsc.subcore_barrier()

    def ocp(j, carry):
      c = j * NS + sid

      @pl.when(c < n_oc)
      def _():
        pltpu.sync_copy(acc_s.at[pl.ds(c * ZC, ZC)],
                        out_hbm.at[cid, pl.ds(c * ZC, ZC)])

      return carry

    lax.fori_loop(0, (n_oc + NS - 1) // NS, ocp, 0)

  return spmm


def _gathers_sc(V, N, Dm, B, L):
  T = B * L
  tpw = T // NW
  C = 80
  nch = tpw // C
  mpw = B // NW

  @functools.partial(
      pl.kernel,
      out_type=(
          jax.ShapeDtypeStruct((T, Dm), jnp.float32),
          jax.ShapeDtypeStruct((B, Dm), jnp.float32),
          jax.ShapeDtypeStruct((B,), jnp.int32),
          jax.ShapeDtypeStruct((B,), jnp.float32),
      ),
      mesh=plsc.VectorSubcoreMesh(core_axis_name="c", subcore_axis_name="s"),
      scratch_types=[
          pltpu.VMEM((C,), jnp.int32),
          pltpu.VMEM((C, Dm), jnp.float32),
          pltpu.VMEM((mpw,), jnp.int32),
          pltpu.VMEM((mpw, Dm), jnp.float32),
          pltpu.VMEM((mpw,), jnp.int32),
          pltpu.VMEM((mpw,), jnp.float32),
          pltpu.SemaphoreType.DMA,
      ],
  )
  def g1(emb_hbm, v2f_hbm, ctx_hbm, miss_hbm,
         ctx_out, base_out, fg_out, mask_out,
         cidx_v, crows_v, midx_v, mrows_v, fg_v, mk_v, sem):
    cid = lax.axis_index("c")
    sid = lax.axis_index("s")
    wid = cid * NS + sid

    def chunk(k, carry):
      base = wid * tpw + k * C
      pltpu.sync_copy(ctx_hbm.at[pl.ds(base, C)], cidx_v)
      pltpu.async_copy(emb_hbm.at[cidx_v], crows_v, sem).wait()
      pltpu.sync_copy(crows_v, ctx_out.at[pl.ds(base, C)])
      return carry

    lax.fori_loop(0, nch, chunk, 0)

    mb = wid * mpw
    pltpu.sync_copy(miss_hbm.at[pl.ds(mb, mpw)], midx_v)
    pltpu.async_copy(emb_hbm.at[midx_v], mrows_v, sem).wait()
    pltpu.sync_copy(mrows_v, base_out.at[pl.ds(mb, mpw)])

    pltpu.async_copy(v2f_hbm.at[midx_v], fg_v, sem).wait()
    for g in range(mpw // LANES):
      fg = fg_v[pl.ds(g * LANES, LANES)]
      mk = jnp.where(fg >= 0, 1.0, 0.0).astype(jnp.float32)
      fgc = jnp.clip(fg, 0, N - 1)
      fg_v[pl.ds(g * LANES, LANES)] = fgc
      mk_v[pl.ds(g * LANES, LANES)] = mk
    pltpu.sync_copy(fg_v, fg_out.at[pl.ds(mb, mpw)])
    pltpu.sync_copy(mk_v, mask_out.at[pl.ds(mb, mpw)])

  return g1


def _gather_graph_sc(N, Dm, B):
  mpw = B // NW
  nq = Dm // LANES

  @functools.partial(
      pl.kernel,
      out_type=jax.ShapeDtypeStruct((B, Dm), jnp.float32),
      mesh=plsc.VectorSubcoreMesh(core_axis_name="c", subcore_axis_name="s"),
      scratch_types=[
          pltpu.VMEM((mpw,), jnp.int32),
          pltpu.VMEM((mpw,), jnp.float32),
          pltpu.VMEM((mpw, Dm), jnp.float32),
          pltpu.SemaphoreType.DMA,
      ],
  )
  def g2(ge_hbm, fg_hbm, mk_hbm, out_hbm, idx_v, mk_v, rows_v, sem):
    cid = lax.axis_index("c")
    sid = lax.axis_index("s")
    wid = cid * NS + sid
    mb = wid * mpw
    pltpu.sync_copy(fg_hbm.at[pl.ds(mb, mpw)], idx_v)
    pltpu.sync_copy(mk_hbm.at[pl.ds(mb, mpw)], mk_v)
    pltpu.async_copy(ge_hbm.at[idx_v], rows_v, sem).wait()

    def row(g, carry):
      mk16 = mk_v[pl.ds(g * LANES, LANES)]
      for j in range(LANES):
        e = g * LANES + j
        mv = jnp.full((LANES,), mk16[j], jnp.float32)
        for q in range(nq):
          sl = pl.ds(q * LANES, LANES)
          rows_v[e, sl] = rows_v[e, sl] * mv
      return carry

    lax.fori_loop(0, mpw // LANES, row, 0)
    pltpu.sync_copy(rows_v, out_hbm.at[pl.ds(mb, mpw)])

  return g2


def _layer_tc(N, Dm, bm):
  def body(p_ref, h_ref, W_ref, g_ref, b_ref, o_ref):
    s = p_ref[0] + p_ref[1]
    y = lax.dot_general(s, W_ref[...], (((1,), (1,)), ((), ())),
                        preferred_element_type=jnp.float32)
    y = jnp.maximum(y, 0.0)
    y = y * (g_ref[...] * BN_INV) + b_ref[...]
    o_ref[...] = h_ref[...] + y

  return pl.pallas_call(
      body,
      grid=(N // bm,),
      in_specs=[
          pl.BlockSpec((NC, bm, Dm), lambda i: (0, i, 0)),
          pl.BlockSpec((bm, Dm), lambda i: (i, 0)),
          pl.BlockSpec((Dm, Dm), lambda i: (0, 0)),
          pl.BlockSpec((1, Dm), lambda i: (0, 0)),
          pl.BlockSpec((1, Dm), lambda i: (0, 0)),
      ],
      out_specs=pl.BlockSpec((bm, Dm), lambda i: (i, 0)),
      out_shape=jax.ShapeDtypeStruct((N, Dm), jnp.float32),
  )


def _attn_tc(B, L, Dm, bb):
  def body(c_ref, w_ref, o_ref):
    c = c_ref[...]
    lg = lax.dot_general(c, w_ref[...], (((2,), (1,)), ((), ())),
                         preferred_element_type=jnp.float32)[:, :, 0]
    m = jnp.max(lg, axis=1, keepdims=True)
    ex = jnp.exp(lg - m)
    a = ex / jnp.sum(ex, axis=1, keepdims=True)
    o_ref[...] = lax.dot_general(a, c, (((1,), (1,)), ((0,), (0,))),
                                 preferred_element_type=jnp.float32)

  return pl.pallas_call(
      body,
      grid=(B // bb,),
      in_specs=[
          pl.BlockSpec((bb, L, Dm), lambda i: (i, 0, 0)),
          pl.BlockSpec((1, Dm), lambda i: (0, 0)),
      ],
      out_specs=pl.BlockSpec((bb, Dm), lambda i: (i, 0)),
      out_shape=jax.ShapeDtypeStruct((B, Dm), jnp.float32),
  )


def _final_tc(B, Dm):
  def body(ctx_ref, base_ref, gp_ref, fW_ref, fb_ref,
           p0W_ref, p0b_ref, p1W_ref, p1b_ref, o_ref):
    x = jnp.concatenate([base_ref[...], gp_ref[...]], axis=1)
    miss = lax.dot_general(x, fW_ref[...], (((1,), (1,)), ((), ())),
                           preferred_element_type=jnp.float32) + fb_ref[...]
    q = jnp.concatenate([ctx_ref[...], miss], axis=1)
    hq = lax.dot_general(q, p0W_ref[...], (((1,), (1,)), ((), ())),
                         preferred_element_type=jnp.float32) + p0b_ref[...]
    hq = jnp.maximum(hq, 0.0)
    o_ref[...] = lax.dot_general(hq, p1W_ref[...], (((1,), (1,)), ((), ())),
                                 preferred_element_type=jnp.float32) + p1b_ref[...]

  return pl.pallas_call(
      body,
      out_shape=jax.ShapeDtypeStruct((B, Dm), jnp.float32),
  )


def kernel(edge_index, edge_weight, ctx_ids, miss_ids, vocab_to_fg, embedding,
           fg_embedding, gc_W0, gc_W1, bn0_gamma, bn0_beta, bn1_gamma,
           bn1_beta, ctx_attn_W, ctx_attn_b, fusion_W, fusion_b, proj0_W,
           proj0_b, proj1_W, proj1_b):
  N, Dm = fg_embedding.shape
  E = edge_weight.shape[0]
  B, L = ctx_ids.shape
  V = embedding.shape[0]

  C = 40
  n_chunks = E // (NW * C)
  src = edge_index[0].reshape(NW, n_chunks, 1, C)
  dst = edge_index[1].reshape(NW, n_chunks, 1, C)
  wgt = edge_weight.reshape(NW, n_chunks, 1, C)

  spmm = _spmm_sc(N, Dm, E)
  layer = _layer_tc(N, Dm, 1000)

  g0 = bn0_gamma.reshape(1, Dm)
  b0 = bn0_beta.reshape(1, Dm)
  g1w = bn1_gamma.reshape(1, Dm)
  b1w = bn1_beta.reshape(1, Dm)

  h1 = layer(spmm(fg_embedding, dst, src, wgt), fg_embedding, gc_W0, g0, b0)
  graph_embs = layer(spmm(h1, dst, src, wgt), h1, gc_W1, g1w, b1w)

  ctx_rows, base_emb, fg_clip, maskf = _gathers_sc(V, N, Dm, B, L)(
      embedding, vocab_to_fg, ctx_ids.reshape(B * L), miss_ids)
  ctx_emb = _attn_tc(B, L, Dm, 128)(ctx_rows.reshape(B, L, Dm), ctx_attn_W)
  gpart = _gather_graph_sc(N, Dm, B)(graph_embs, fg_clip, maskf)

  query = _final_tc(B, Dm)(
      ctx_emb, base_emb, gpart,
      fusion_W, fusion_b.reshape(1, Dm),
      proj0_W, proj0_b.reshape(1, 2 * Dm),
      proj1_W, proj1_b.reshape(1, Dm))

  return (query, graph_embs)


# speedup vs baseline: 7.1150x; 1.0399x over previous
import functools
import math

import jax
import jax.numpy as jnp
from jax import lax
from jax.experimental import pallas as pl
from jax.experimental.pallas import tpu as pltpu
from jax.experimental.pallas import tpu_sc as plsc

NC = 2
NS = 16
LANES = 16
NW = NC * NS
BN_INV = 1.0 / math.sqrt(1.0 + 1e-5)


def _spmm_sc(N, Dm, E):
  C = 40
  n_chunks = E // (NW * C)
  NB = 4
  NI = 8
  ZC = 200
  n_oc = N // ZC
  ZCz = C
  n_zc = N // ZCz
  nq = Dm // LANES

  @functools.partial(
      pl.kernel,
      out_type=jax.ShapeDtypeStruct((NC, N, Dm), jnp.float32),
      mesh=plsc.VectorSubcoreMesh(core_axis_name="c", subcore_axis_name="s"),
      scratch_types=[
          pltpu.VMEM((NI, 1, C), jnp.int32),
          pltpu.VMEM((NI, 1, C), jnp.int32),
          pltpu.VMEM((NI, 1, C), jnp.float32),
          pltpu.VMEM((NB, C, Dm), jnp.float32),
          pltpu.VMEM_SHARED((N, Dm), jnp.float32),
          [pltpu.SemaphoreType.DMA] * NB,
          [pltpu.SemaphoreType.DMA] * NB,
          [pltpu.SemaphoreType.DMA] * NI,
      ],
  )
  def spmm(h_hbm, dst_hbm, src_hbm, w_hbm, out_hbm,
           dstb, srcb, wb, rows_v, acc_s, gsems, ssems, isems):
    cid = lax.axis_index("c")
    sid = lax.axis_index("s")
    wid = cid * NS + sid

    zv = jnp.zeros((LANES,), jnp.float32)

    def zrow(r, carry):
      for q in range(nq):
        rows_v[0, r, pl.ds(q * LANES, LANES)] = zv
      return carry

    lax.fori_loop(0, C, zrow, 0)

    def zcp(j, carry):
      c = j * NS + sid

      @pl.when(c < n_zc)
      def _():
        pltpu.sync_copy(rows_v.at[0], acc_s.at[pl.ds(c * ZCz, ZCz)])

      return carry

    lax.fori_loop(0, (n_zc + NS - 1) // NS, zcp, 0)
    plsc.subcore_barrier()

    def issue_idx(k, i):
      pltpu.async_copy(dst_hbm.at[wid, k], dstb.at[i], isems[i])
      pltpu.async_copy(src_hbm.at[wid, k], srcb.at[i], isems[i])
      pltpu.async_copy(w_hbm.at[wid, k], wb.at[i], isems[i])

    def wait_idx(i):
      pltpu.make_async_copy(dst_hbm.at[wid, 0], dstb.at[i], isems[i]).wait()
      pltpu.make_async_copy(src_hbm.at[wid, 0], srcb.at[i], isems[i]).wait()
      pltpu.make_async_copy(w_hbm.at[wid, 0], wb.at[i], isems[i]).wait()

    def issue_gather(i, b):
      pltpu.async_copy(h_hbm.at[dstb.at[i, 0]], rows_v.at[b], gsems[b])

    def wait_gather(b):
      pltpu.make_async_copy(h_hbm.at[dstb.at[0, 0]], rows_v.at[b],
                            gsems[b]).wait()

    def issue_scatter(i, b):
      pltpu.async_copy(rows_v.at[b], acc_s.at[srcb.at[i, 0]], ssems[b],
                       add=True)

    def wait_scatter(b):
      pltpu.make_async_copy(rows_v.at[b], acc_s.at[srcb.at[0, 0]],
                            ssems[b]).wait()

    def scale(i, b):
      def grp(g, c2):
        w16 = wb[i, 0, pl.ds(g * LANES, LANES)]
        for j in range(LANES):
          e = g * LANES + j
          wv = jnp.full((LANES,), w16[j], jnp.float32)
          for q in range(nq):
            sl = pl.ds(q * LANES, LANES)
            rows_v[b, e, sl] = rows_v[b, e, sl] * wv
        return c2

      lax.fori_loop(0, C // LANES, grp, 0)
      if C % LANES:
        off = C - LANES
        w16 = wb[i, 0, pl.ds(off, LANES)]
        for j in range(LANES - (C % LANES), LANES):
          e = off + j
          wv = jnp.full((LANES,), w16[j], jnp.float32)
          for q in range(nq):
            sl = pl.ds(q * LANES, LANES)
            rows_v[b, e, sl] = rows_v[b, e, sl] * wv

    def step(k, u):
      k = jnp.int32(k)
      b = u % NB
      b2 = (u + 2) % NB
      i2 = (u + 2) % NI
      i4 = (u + 4) % NI

      @pl.when(k >= 2)
      def _():
        wait_scatter(b2)

      @pl.when(k + 2 < n_chunks)
      def _():
        wait_idx(i2)
        issue_gather(i2, b2)

      @pl.when(k + 4 < n_chunks)
      def _():
        issue_idx(k + 4, i4)

      wait_gather(b)
      scale(u % NI, b)
      issue_scatter(u % NI, b)

    for j in range(4):
      issue_idx(j, j)
    wait_idx(0)
    issue_gather(0, 0)
    wait_idx(1)
    issue_gather(1, 1)

    n_main = (n_chunks // NI) * NI

    def outer(g, carry):
      for u in range(NI):
        step(g * NI + u, u)
      return carry

    lax.fori_loop(0, n_main // NI, outer, 0)

    for k in range(n_main, n_chunks):
      step(k, k % NI)

    for k in range(n_chunks - 2, n_chunks):
      wait_scatter(k % NB)
    plsc.subcore_barrier()

    def ocp(j, carry):
      c = j * NS + sid

      @pl.when(c < n_oc)
      def _():
        pltpu.sync_copy(acc_s.at[pl.ds(c * ZC, ZC)],
                        out_hbm.at[cid, pl.ds(c * ZC, ZC)])

      return carry

    lax.fori_loop(0, (n_oc + NS - 1) // NS, ocp, 0)

  return spmm


def _gathers_sc(V, N, Dm, B, L):
  T = B * L
  tpw = T // NW
  C = 128
  mpw = B // NW
  chunks = []
  off = 0
  while off < tpw:
    ln = min(C, tpw - off)
    chunks.append((off, ln))
    off += ln
  nch = len(chunks)
  NBG = 4

  @functools.partial(
      pl.kernel,
      out_type=(
          jax.ShapeDtypeStruct((T, Dm), jnp.float32),
          jax.ShapeDtypeStruct((B, Dm), jnp.float32),
          jax.ShapeDtypeStruct((B,), jnp.int32),
          jax.ShapeDtypeStruct((B,), jnp.float32),
      ),
      mesh=plsc.VectorSubcoreMesh(core_axis_name="c", subcore_axis_name="s"),
      scratch_types=[
          pltpu.VMEM((tpw,), jnp.int32),
          pltpu.VMEM((NBG, C, Dm), jnp.float32),
          pltpu.VMEM((mpw,), jnp.int32),
          pltpu.VMEM((mpw, Dm), jnp.float32),
          pltpu.VMEM((mpw,), jnp.int32),
          pltpu.VMEM((mpw,), jnp.float32),
          [pltpu.SemaphoreType.DMA] * NBG,
          [pltpu.SemaphoreType.DMA] * NBG,
          pltpu.SemaphoreType.DMA,
      ],
  )
  def g1(emb_hbm, v2f_hbm, ctx_hbm, miss_hbm,
         ctx_out, base_out, fg_out, mask_out,
         cidx_v, crows_v, midx_v, mrows_v, fg_v, mk_v, gsems, osems, sem):
    cid = lax.axis_index("c")
    sid = lax.axis_index("s")
    wid = cid * NS + sid
    base0 = wid * tpw

    pltpu.sync_copy(ctx_hbm.at[pl.ds(base0, tpw)], cidx_v)

    def issue_gather(k):
      o, ln = chunks[k]
      b = k % NBG
      pltpu.async_copy(emb_hbm.at[cidx_v.at[pl.ds(o, ln)]],
                       crows_v.at[b, pl.ds(0, ln)], gsems[b])

    def wait_gather(k):
      o, ln = chunks[k]
      b = k % NBG
      pltpu.make_async_copy(emb_hbm.at[cidx_v.at[pl.ds(o, ln)]],
                            crows_v.at[b, pl.ds(0, ln)], gsems[b]).wait()

    def issue_out(k):
      o, ln = chunks[k]
      b = k % NBG
      pltpu.async_copy(crows_v.at[b, pl.ds(0, ln)],
                       ctx_out.at[pl.ds(base0 + o, ln)], osems[b])

    def wait_out(k):
      o, ln = chunks[k]
      b = k % NBG
      pltpu.make_async_copy(crows_v.at[b, pl.ds(0, ln)],
                            ctx_out.at[pl.ds(base0 + o, ln)], osems[b]).wait()

    issue_gather(0)
    if nch > 1:
      issue_gather(1)
    for k in range(nch):
      wait_gather(k)
      issue_out(k)
      if k + 2 < nch:
        if k >= 2:
          wait_out(k - 2)
        issue_gather(k + 2)
    for k in range(max(0, nch - 4), nch):
      wait_out(k)

    mb = wid * mpw
    pltpu.sync_copy(miss_hbm.at[pl.ds(mb, mpw)], midx_v)
    pltpu.async_copy(emb_hbm.at[midx_v], mrows_v, sem).wait()
    pltpu.sync_copy(mrows_v, base_out.at[pl.ds(mb, mpw)])

    pltpu.async_copy(v2f_hbm.at[midx_v], fg_v, sem).wait()
    for g in range(mpw // LANES):
      fg = fg_v[pl.ds(g * LANES, LANES)]
      mk = jnp.where(fg >= 0, 1.0, 0.0).astype(jnp.float32)
      fgc = jnp.clip(fg, 0, N - 1)
      fg_v[pl.ds(g * LANES, LANES)] = fgc
      mk_v[pl.ds(g * LANES, LANES)] = mk
    pltpu.sync_copy(fg_v, fg_out.at[pl.ds(mb, mpw)])
    pltpu.sync_copy(mk_v, mask_out.at[pl.ds(mb, mpw)])

  return g1


def _gather_graph_sc(N, Dm, B):
  mpw = B // NW
  nq = Dm // LANES

  @functools.partial(
      pl.kernel,
      out_type=jax.ShapeDtypeStruct((B, Dm), jnp.float32),
      mesh=plsc.VectorSubcoreMesh(core_axis_name="c", subcore_axis_name="s"),
      scratch_types=[
          pltpu.VMEM((mpw,), jnp.int32),
          pltpu.VMEM((mpw,), jnp.float32),
          pltpu.VMEM((mpw, Dm), jnp.float32),
          pltpu.SemaphoreType.DMA,
      ],
  )
  def g2(ge_hbm, fg_hbm, mk_hbm, out_hbm, idx_v, mk_v, rows_v, sem):
    cid = lax.axis_index("c")
    sid = lax.axis_index("s")
    wid = cid * NS + sid
    mb = wid * mpw
    pltpu.sync_copy(fg_hbm.at[pl.ds(mb, mpw)], idx_v)
    pltpu.sync_copy(mk_hbm.at[pl.ds(mb, mpw)], mk_v)
    pltpu.async_copy(ge_hbm.at[idx_v], rows_v, sem).wait()

    def row(g, carry):
      mk16 = mk_v[pl.ds(g * LANES, LANES)]
      for j in range(LANES):
        e = g * LANES + j
        mv = jnp.full((LANES,), mk16[j], jnp.float32)
        for q in range(nq):
          sl = pl.ds(q * LANES, LANES)
          rows_v[e, sl] = rows_v[e, sl] * mv
      return carry

    lax.fori_loop(0, mpw // LANES, row, 0)
    pltpu.sync_copy(rows_v, out_hbm.at[pl.ds(mb, mpw)])

  return g2


def _layer_tc(N, Dm, bm):
  def body(p_ref, h_ref, W_ref, g_ref, b_ref, o_ref):
    s = p_ref[0] + p_ref[1]
    y = lax.dot_general(s, W_ref[...], (((1,), (1,)), ((), ())),
                        preferred_element_type=jnp.float32)
    y = jnp.maximum(y, 0.0)
    y = y * (g_ref[...] * BN_INV) + b_ref[...]
    o_ref[...] = h_ref[...] + y

  return pl.pallas_call(
      body,
      grid=(N // bm,),
      in_specs=[
          pl.BlockSpec((NC, bm, Dm), lambda i: (0, i, 0)),
          pl.BlockSpec((bm, Dm), lambda i: (i, 0)),
          pl.BlockSpec((Dm, Dm), lambda i: (0, 0)),
          pl.BlockSpec((1, Dm), lambda i: (0, 0)),
          pl.BlockSpec((1, Dm), lambda i: (0, 0)),
      ],
      out_specs=pl.BlockSpec((bm, Dm), lambda i: (i, 0)),
      out_shape=jax.ShapeDtypeStruct((N, Dm), jnp.float32),
  )


def _head_tc(B, L, Dm, bb):
  def body(c_ref, aw_ref, base_ref, gp_ref, fW_ref, fb_ref,
           p0W_ref, p0b_ref, p1W_ref, p1b_ref, o_ref):
    c = c_ref[...]
    lg = lax.dot_general(c, aw_ref[...], (((2,), (1,)), ((), ())),
                         preferred_element_type=jnp.float32)[:, :, 0]
    m = jnp.max(lg, axis=1, keepdims=True)
    ex = jnp.exp(lg - m)
    a = ex / jnp.sum(ex, axis=1, keepdims=True)
    ctx_emb = lax.dot_general(a, c, (((1,), (1,)), ((0,), (0,))),
                              preferred_element_type=jnp.float32)
    x = jnp.concatenate([base_ref[...], gp_ref[...]], axis=1)
    miss = lax.dot_general(x, fW_ref[...], (((1,), (1,)), ((), ())),
                           preferred_element_type=jnp.float32) + fb_ref[...]
    q = jnp.concatenate([ctx_emb, miss], axis=1)
    hq = lax.dot_general(q, p0W_ref[...], (((1,), (1,)), ((), ())),
                         preferred_element_type=jnp.float32) + p0b_ref[...]
    hq = jnp.maximum(hq, 0.0)
    o_ref[...] = lax.dot_general(hq, p1W_ref[...], (((1,), (1,)), ((), ())),
                                 preferred_element_type=jnp.float32) + p1b_ref[...]

  full = lambda *s: pl.BlockSpec(s, lambda i: tuple(0 for _ in s))
  return pl.pallas_call(
      body,
      grid=(B // bb,),
      in_specs=[
          pl.BlockSpec((bb, L, Dm), lambda i: (i, 0, 0)),
          full(1, Dm),
          pl.BlockSpec((bb, Dm), lambda i: (i, 0)),
          pl.BlockSpec((bb, Dm), lambda i: (i, 0)),
          full(Dm, 2 * Dm),
          full(1, Dm),
          full(2 * Dm, 2 * Dm),
          full(1, 2 * Dm),
          full(Dm, 2 * Dm),
          full(1, Dm),
      ],
      out_specs=pl.BlockSpec((bb, Dm), lambda i: (i, 0)),
      out_shape=jax.ShapeDtypeStruct((B, Dm), jnp.float32),
  )


def kernel(edge_index, edge_weight, ctx_ids, miss_ids, vocab_to_fg, embedding,
           fg_embedding, gc_W0, gc_W1, bn0_gamma, bn0_beta, bn1_gamma,
           bn1_beta, ctx_attn_W, ctx_attn_b, fusion_W, fusion_b, proj0_W,
           proj0_b, proj1_W, proj1_b):
  N, Dm = fg_embedding.shape
  E = edge_weight.shape[0]
  B, L = ctx_ids.shape
  V = embedding.shape[0]

  C = 40
  n_chunks = E // (NW * C)
  src = edge_index[0].reshape(NW, n_chunks, 1, C)
  dst = edge_index[1].reshape(NW, n_chunks, 1, C)
  wgt = edge_weight.reshape(NW, n_chunks, 1, C)

  spmm = _spmm_sc(N, Dm, E)
  layer = _layer_tc(N, Dm, 1000)

  g0 = bn0_gamma.reshape(1, Dm)
  b0 = bn0_beta.reshape(1, Dm)
  g1w = bn1_gamma.reshape(1, Dm)
  b1w = bn1_beta.reshape(1, Dm)

  h1 = layer(spmm(fg_embedding, dst, src, wgt), fg_embedding, gc_W0, g0, b0)
  graph_embs = layer(spmm(h1, dst, src, wgt), h1, gc_W1, g1w, b1w)

  ctx_rows, base_emb, fg_clip, maskf = _gathers_sc(V, N, Dm, B, L)(
      embedding, vocab_to_fg, ctx_ids.reshape(B * L), miss_ids)
  gpart = _gather_graph_sc(N, Dm, B)(graph_embs, fg_clip, maskf)

  query = _head_tc(B, L, Dm, 128)(
      ctx_rows.reshape(B, L, Dm), ctx_attn_W, base_emb, gpart,
      fusion_W, fusion_b.reshape(1, Dm),
      proj0_W, proj0_b.reshape(1, 2 * Dm),
      proj1_W, proj1_b.reshape(1, Dm))

  return (query, graph_embs)


# speedup vs baseline: 8.0547x; 1.1321x over previous
import functools
import math

import jax
import jax.numpy as jnp
from jax import lax
from jax.experimental import pallas as pl
from jax.experimental.pallas import tpu as pltpu
from jax.experimental.pallas import tpu_sc as plsc

NC = 2
NS = 16
LANES = 16
NW = NC * NS
BN_INV = 1.0 / math.sqrt(1.0 + 1e-5)


def _spmm_sc(N, Dm, E):
  C = 40
  n_chunks = E // (NW * C)
  NB = 4
  NI = 8
  ZC = 200
  n_oc = N // ZC
  ZCz = C
  n_zc = N // ZCz
  nq = Dm // LANES

  @functools.partial(
      pl.kernel,
      out_type=jax.ShapeDtypeStruct((NC, N, Dm), jnp.float32),
      mesh=plsc.VectorSubcoreMesh(core_axis_name="c", subcore_axis_name="s"),
      scratch_types=[
          pltpu.VMEM((NI, 1, C), jnp.int32),
          pltpu.VMEM((NI, 1, C), jnp.int32),
          pltpu.VMEM((NI, 1, C), jnp.float32),
          pltpu.VMEM((NB, C, Dm), jnp.float32),
          pltpu.VMEM_SHARED((N, Dm), jnp.float32),
          [pltpu.SemaphoreType.DMA] * NB,
          [pltpu.SemaphoreType.DMA] * NB,
          [pltpu.SemaphoreType.DMA] * NI,
      ],
  )
  def spmm(h_hbm, dst_hbm, src_hbm, w_hbm, out_hbm,
           dstb, srcb, wb, rows_v, acc_s, gsems, ssems, isems):
    cid = lax.axis_index("c")
    sid = lax.axis_index("s")
    wid = cid * NS + sid

    zv = jnp.zeros((LANES,), jnp.float32)

    def zrow(r, carry):
      for q in range(nq):
        rows_v[0, r, pl.ds(q * LANES, LANES)] = zv
      return carry

    lax.fori_loop(0, C, zrow, 0)

    def zcp(j, carry):
      c = j * NS + sid

      @pl.when(c < n_zc)
      def _():
        pltpu.sync_copy(rows_v.at[0], acc_s.at[pl.ds(c * ZCz, ZCz)])

      return carry

    lax.fori_loop(0, (n_zc + NS - 1) // NS, zcp, 0)
    plsc.subcore_barrier()

    ebase = wid * (E // NW)

    def issue_idx(k, i):
      sl = pl.ds(ebase + k * C, C)
      pltpu.async_copy(dst_hbm.at[sl], dstb.at[i, 0], isems[i])
      pltpu.async_copy(src_hbm.at[sl], srcb.at[i, 0], isems[i])
      pltpu.async_copy(w_hbm.at[sl], wb.at[i, 0], isems[i])

    def wait_idx(i):
      sl = pl.ds(ebase, C)
      pltpu.make_async_copy(dst_hbm.at[sl], dstb.at[i, 0], isems[i]).wait()
      pltpu.make_async_copy(src_hbm.at[sl], srcb.at[i, 0], isems[i]).wait()
      pltpu.make_async_copy(w_hbm.at[sl], wb.at[i, 0], isems[i]).wait()

    def issue_gather(i, b):
      pltpu.async_copy(h_hbm.at[dstb.at[i, 0]], rows_v.at[b], gsems[b])

    def wait_gather(b):
      pltpu.make_async_copy(h_hbm.at[dstb.at[0, 0]], rows_v.at[b],
                            gsems[b]).wait()

    def issue_scatter(i, b):
      pltpu.async_copy(rows_v.at[b], acc_s.at[srcb.at[i, 0]], ssems[b],
                       add=True)

    def wait_scatter(b):
      pltpu.make_async_copy(rows_v.at[b], acc_s.at[srcb.at[0, 0]],
                            ssems[b]).wait()

    def scale(i, b):
      def grp(g, c2):
        w16 = wb[i, 0, pl.ds(g * LANES, LANES)]
        for j in range(LANES):
          e = g * LANES + j
          wv = jnp.full((LANES,), w16[j], jnp.float32)
          for q in range(nq):
            sl = pl.ds(q * LANES, LANES)
            rows_v[b, e, sl] = rows_v[b, e, sl] * wv
        return c2

      lax.fori_loop(0, C // LANES, grp, 0)
      if C % LANES:
        off = C - LANES
        w16 = wb[i, 0, pl.ds(off, LANES)]
        for j in range(LANES - (C % LANES), LANES):
          e = off + j
          wv = jnp.full((LANES,), w16[j], jnp.float32)
          for q in range(nq):
            sl = pl.ds(q * LANES, LANES)
            rows_v[b, e, sl] = rows_v[b, e, sl] * wv

    def step(k, u):
      k = jnp.int32(k)
      b = u % NB
      b2 = (u + 2) % NB
      i2 = (u + 2) % NI
      i4 = (u + 4) % NI

      @pl.when(k >= 2)
      def _():
        wait_scatter(b2)

      @pl.when(k + 2 < n_chunks)
      def _():
        wait_idx(i2)
        issue_gather(i2, b2)

      @pl.when(k + 4 < n_chunks)
      def _():
        issue_idx(k + 4, i4)

      wait_gather(b)
      scale(u % NI, b)
      issue_scatter(u % NI, b)

    for j in range(4):
      issue_idx(j, j)
    wait_idx(0)
    issue_gather(0, 0)
    wait_idx(1)
    issue_gather(1, 1)

    n_main = (n_chunks // NI) * NI

    def outer(g, carry):
      for u in range(NI):
        step(g * NI + u, u)
      return carry

    lax.fori_loop(0, n_main // NI, outer, 0)

    for k in range(n_main, n_chunks):
      step(k, k % NI)

    for k in range(n_chunks - 2, n_chunks):
      wait_scatter(k % NB)
    plsc.subcore_barrier()

    def ocp(j, carry):
      c = j * NS + sid

      @pl.when(c < n_oc)
      def _():
        pltpu.sync_copy(acc_s.at[pl.ds(c * ZC, ZC)],
                        out_hbm.at[cid, pl.ds(c * ZC, ZC)])

      return carry

    lax.fori_loop(0, (n_oc + NS - 1) // NS, ocp, 0)

  return spmm


def _gathers_sc(V, N, Dm, B, L):
  T = B * L
  tpw = T // NW
  C = 128
  mpw = B // NW
  chunks = []
  off = 0
  while off < tpw:
    ln = min(C, tpw - off)
    chunks.append((off, ln))
    off += ln
  nch = len(chunks)
  NBG = 4

  @functools.partial(
      pl.kernel,
      out_type=(
          jax.ShapeDtypeStruct((T, Dm), jnp.float32),
          jax.ShapeDtypeStruct((B, Dm), jnp.float32),
          jax.ShapeDtypeStruct((B,), jnp.int32),
          jax.ShapeDtypeStruct((B,), jnp.float32),
      ),
      mesh=plsc.VectorSubcoreMesh(core_axis_name="c", subcore_axis_name="s"),
      scratch_types=[
          pltpu.VMEM((tpw,), jnp.int32),
          pltpu.VMEM((NBG, C, Dm), jnp.float32),
          pltpu.VMEM((mpw,), jnp.int32),
          pltpu.VMEM((mpw, Dm), jnp.float32),
          pltpu.VMEM((mpw,), jnp.int32),
          pltpu.VMEM((mpw,), jnp.float32),
          [pltpu.SemaphoreType.DMA] * NBG,
          [pltpu.SemaphoreType.DMA] * NBG,
          pltpu.SemaphoreType.DMA,
      ],
  )
  def g1(emb_hbm, v2f_hbm, ctx_hbm, miss_hbm,
         ctx_out, base_out, fg_out, mask_out,
         cidx_v, crows_v, midx_v, mrows_v, fg_v, mk_v, gsems, osems, sem):
    cid = lax.axis_index("c")
    sid = lax.axis_index("s")
    wid = cid * NS + sid
    base0 = wid * tpw

    pltpu.sync_copy(ctx_hbm.at[pl.ds(base0, tpw)], cidx_v)

    def issue_gather(k):
      o, ln = chunks[k]
      b = k % NBG
      pltpu.async_copy(emb_hbm.at[cidx_v.at[pl.ds(o, ln)]],
                       crows_v.at[b, pl.ds(0, ln)], gsems[b])

    def wait_gather(k):
      o, ln = chunks[k]
      b = k % NBG
      pltpu.make_async_copy(emb_hbm.at[cidx_v.at[pl.ds(o, ln)]],
                            crows_v.at[b, pl.ds(0, ln)], gsems[b]).wait()

    def issue_out(k):
      o, ln = chunks[k]
      b = k % NBG
      pltpu.async_copy(crows_v.at[b, pl.ds(0, ln)],
                       ctx_out.at[pl.ds(base0 + o, ln)], osems[b])

    def wait_out(k):
      o, ln = chunks[k]
      b = k % NBG
      pltpu.make_async_copy(crows_v.at[b, pl.ds(0, ln)],
                            ctx_out.at[pl.ds(base0 + o, ln)], osems[b]).wait()

    issue_gather(0)
    if nch > 1:
      issue_gather(1)
    for k in range(nch):
      wait_gather(k)
      issue_out(k)
      if k + 2 < nch:
        if k >= 2:
          wait_out(k - 2)
        issue_gather(k + 2)
    for k in range(max(0, nch - 4), nch):
      wait_out(k)

    mb = wid * mpw
    pltpu.sync_copy(miss_hbm.at[pl.ds(mb, mpw)], midx_v)
    pltpu.async_copy(emb_hbm.at[midx_v], mrows_v, sem).wait()
    pltpu.sync_copy(mrows_v, base_out.at[pl.ds(mb, mpw)])

    pltpu.async_copy(v2f_hbm.at[midx_v], fg_v, sem).wait()
    for g in range(mpw // LANES):
      fg = fg_v[pl.ds(g * LANES, LANES)]
      mk = jnp.where(fg >= 0, 1.0, 0.0).astype(jnp.float32)
      fgc = jnp.clip(fg, 0, N - 1)
      fg_v[pl.ds(g * LANES, LANES)] = fgc
      mk_v[pl.ds(g * LANES, LANES)] = mk
    pltpu.sync_copy(fg_v, fg_out.at[pl.ds(mb, mpw)])
    pltpu.sync_copy(mk_v, mask_out.at[pl.ds(mb, mpw)])

  return g1


def _gather_graph_sc(N, Dm, B):
  mpw = B // NW
  nq = Dm // LANES

  @functools.partial(
      pl.kernel,
      out_type=jax.ShapeDtypeStruct((B, Dm), jnp.float32),
      mesh=plsc.VectorSubcoreMesh(core_axis_name="c", subcore_axis_name="s"),
      scratch_types=[
          pltpu.VMEM((mpw,), jnp.int32),
          pltpu.VMEM((mpw,), jnp.float32),
          pltpu.VMEM((mpw, Dm), jnp.float32),
          pltpu.SemaphoreType.DMA,
      ],
  )
  def g2(ge_hbm, fg_hbm, mk_hbm, out_hbm, idx_v, mk_v, rows_v, sem):
    cid = lax.axis_index("c")
    sid = lax.axis_index("s")
    wid = cid * NS + sid
    mb = wid * mpw
    pltpu.sync_copy(fg_hbm.at[pl.ds(mb, mpw)], idx_v)
    pltpu.sync_copy(mk_hbm.at[pl.ds(mb, mpw)], mk_v)
    pltpu.async_copy(ge_hbm.at[idx_v], rows_v, sem).wait()

    def row(g, carry):
      mk16 = mk_v[pl.ds(g * LANES, LANES)]
      for j in range(LANES):
        e = g * LANES + j
        mv = jnp.full((LANES,), mk16[j], jnp.float32)
        for q in range(nq):
          sl = pl.ds(q * LANES, LANES)
          rows_v[e, sl] = rows_v[e, sl] * mv
      return carry

    lax.fori_loop(0, mpw // LANES, row, 0)
    pltpu.sync_copy(rows_v, out_hbm.at[pl.ds(mb, mpw)])

  return g2


def _layer_tc(N, Dm, bm):
  def body(p_ref, h_ref, W_ref, g_ref, b_ref, o_ref):
    s = p_ref[0] + p_ref[1]
    y = lax.dot_general(s, W_ref[...], (((1,), (1,)), ((), ())),
                        preferred_element_type=jnp.float32)
    y = jnp.maximum(y, 0.0)
    y = y * (g_ref[...] * BN_INV) + b_ref[...]
    o_ref[...] = h_ref[...] + y

  return pl.pallas_call(
      body,
      grid=(N // bm,),
      in_specs=[
          pl.BlockSpec((NC, bm, Dm), lambda i: (0, i, 0)),
          pl.BlockSpec((bm, Dm), lambda i: (i, 0)),
          pl.BlockSpec((Dm, Dm), lambda i: (0, 0)),
          pl.BlockSpec((1, Dm), lambda i: (0, 0)),
          pl.BlockSpec((1, Dm), lambda i: (0, 0)),
      ],
      out_specs=pl.BlockSpec((bm, Dm), lambda i: (i, 0)),
      out_shape=jax.ShapeDtypeStruct((N, Dm), jnp.float32),
  )


def _head_tc(B, L, Dm, bb):
  def body(c_ref, aw_ref, base_ref, gp_ref, fW_ref, fb_ref,
           p0W_ref, p0b_ref, p1W_ref, p1b_ref, o_ref):
    c = c_ref[...].reshape(bb, L, Dm)
    lg = lax.dot_general(c, aw_ref[...], (((2,), (1,)), ((), ())),
                         preferred_element_type=jnp.float32)[:, :, 0]
    m = jnp.max(lg, axis=1, keepdims=True)
    ex = jnp.exp(lg - m)
    a = ex / jnp.sum(ex, axis=1, keepdims=True)
    ctx_emb = lax.dot_general(a, c, (((1,), (1,)), ((0,), (0,))),
                              preferred_element_type=jnp.float32)
    x = jnp.concatenate([base_ref[...], gp_ref[...]], axis=1)
    miss = lax.dot_general(x, fW_ref[...], (((1,), (1,)), ((), ())),
                           preferred_element_type=jnp.float32) + fb_ref[...]
    q = jnp.concatenate([ctx_emb, miss], axis=1)
    hq = lax.dot_general(q, p0W_ref[...], (((1,), (1,)), ((), ())),
                         preferred_element_type=jnp.float32) + p0b_ref[...]
    hq = jnp.maximum(hq, 0.0)
    o_ref[...] = lax.dot_general(hq, p1W_ref[...], (((1,), (1,)), ((), ())),
                                 preferred_element_type=jnp.float32) + p1b_ref[...]

  full = lambda *s: pl.BlockSpec(s, lambda i: tuple(0 for _ in s))
  return pl.pallas_call(
      body,
      grid=(B // bb,),
      in_specs=[
          pl.BlockSpec((bb * L, Dm), lambda i: (i, 0)),
          full(1, Dm),
          pl.BlockSpec((bb, Dm), lambda i: (i, 0)),
          pl.BlockSpec((bb, Dm), lambda i: (i, 0)),
          full(Dm, 2 * Dm),
          full(1, Dm),
          full(2 * Dm, 2 * Dm),
          full(1, 2 * Dm),
          full(Dm, 2 * Dm),
          full(1, Dm),
      ],
      out_specs=pl.BlockSpec((bb, Dm), lambda i: (i, 0)),
      out_shape=jax.ShapeDtypeStruct((B, Dm), jnp.float32),
  )


def kernel(edge_index, edge_weight, ctx_ids, miss_ids, vocab_to_fg, embedding,
           fg_embedding, gc_W0, gc_W1, bn0_gamma, bn0_beta, bn1_gamma,
           bn1_beta, ctx_attn_W, ctx_attn_b, fusion_W, fusion_b, proj0_W,
           proj0_b, proj1_W, proj1_b):
  N, Dm = fg_embedding.shape
  E = edge_weight.shape[0]
  B, L = ctx_ids.shape
  V = embedding.shape[0]

  spmm = _spmm_sc(N, Dm, E)
  layer = _layer_tc(N, Dm, 1000)

  g0 = bn0_gamma.reshape(1, Dm)
  b0 = bn0_beta.reshape(1, Dm)
  g1w = bn1_gamma.reshape(1, Dm)
  b1w = bn1_beta.reshape(1, Dm)

  src_flat = edge_index[0]
  dst_flat = edge_index[1]
  h1 = layer(spmm(fg_embedding, dst_flat, src_flat, edge_weight),
             fg_embedding, gc_W0, g0, b0)
  graph_embs = layer(spmm(h1, dst_flat, src_flat, edge_weight),
                     h1, gc_W1, g1w, b1w)

  ctx_rows, base_emb, fg_clip, maskf = _gathers_sc(V, N, Dm, B, L)(
      embedding, vocab_to_fg, ctx_ids.reshape(B * L), miss_ids)
  gpart = _gather_graph_sc(N, Dm, B)(graph_embs, fg_clip, maskf)

  query = _head_tc(B, L, Dm, 128)(
      ctx_rows, ctx_attn_W, base_emb, gpart,
      fusion_W, fusion_b.reshape(1, Dm),
      proj0_W, proj0_b.reshape(1, 2 * Dm),
      proj1_W, proj1_b.reshape(1, Dm))

  return (query, graph_embs)


# speedup vs baseline: 9.1986x; 1.1420x over previous
import functools
import math

import jax
import jax.numpy as jnp
from jax import lax
from jax.experimental import pallas as pl
from jax.experimental.pallas import tpu as pltpu
from jax.experimental.pallas import tpu_sc as plsc

NC = 2
NS = 16
LANES = 16
NW = NC * NS
BN_INV = 1.0 / math.sqrt(1.0 + 1e-5)


def _spmm_sc(N, Dm, E):
  C = 40
  n_chunks = E // (NW * C)
  NB = 4
  NI = 8
  ZC = 200
  n_oc = N // ZC
  ZCz = C
  n_zc = N // ZCz
  nq = Dm // LANES

  @functools.partial(
      pl.kernel,
      out_type=jax.ShapeDtypeStruct((NC, N, Dm), jnp.float32),
      mesh=plsc.VectorSubcoreMesh(core_axis_name="c", subcore_axis_name="s"),
      scratch_types=[
          pltpu.VMEM((NI, 1, C), jnp.int32),
          pltpu.VMEM((NI, 1, C), jnp.int32),
          pltpu.VMEM((NI, 1, C), jnp.float32),
          pltpu.VMEM((NB, C, Dm), jnp.float32),
          pltpu.VMEM_SHARED((N, Dm), jnp.float32),
          [pltpu.SemaphoreType.DMA] * NB,
          [pltpu.SemaphoreType.DMA] * NB,
          [pltpu.SemaphoreType.DMA] * NI,
      ],
  )
  def spmm(h_hbm, dst_hbm, src_hbm, w_hbm, out_hbm,
           dstb, srcb, wb, rows_v, acc_s, gsems, ssems, isems):
    cid = lax.axis_index("c")
    sid = lax.axis_index("s")
    wid = cid * NS + sid

    zv = jnp.zeros((LANES,), jnp.float32)

    def zrow(r, carry):
      for q in range(nq):
        rows_v[0, r, pl.ds(q * LANES, LANES)] = zv
      return carry

    lax.fori_loop(0, C, zrow, 0)

    def zcp(j, carry):
      c = j * NS + sid

      @pl.when(c < n_zc)
      def _():
        pltpu.sync_copy(rows_v.at[0], acc_s.at[pl.ds(c * ZCz, ZCz)])

      return carry

    lax.fori_loop(0, (n_zc + NS - 1) // NS, zcp, 0)
    plsc.subcore_barrier()

    ebase = wid * (E // NW)

    def issue_idx(k, i):
      sl = pl.ds(ebase + k * C, C)
      pltpu.async_copy(dst_hbm.at[sl], dstb.at[i, 0], isems[i])
      pltpu.async_copy(src_hbm.at[sl], srcb.at[i, 0], isems[i])
      pltpu.async_copy(w_hbm.at[sl], wb.at[i, 0], isems[i])

    def wait_idx(i):
      sl = pl.ds(ebase, C)
      pltpu.make_async_copy(dst_hbm.at[sl], dstb.at[i, 0], isems[i]).wait()
      pltpu.make_async_copy(src_hbm.at[sl], srcb.at[i, 0], isems[i]).wait()
      pltpu.make_async_copy(w_hbm.at[sl], wb.at[i, 0], isems[i]).wait()

    def issue_gather(i, b):
      pltpu.async_copy(h_hbm.at[dstb.at[i, 0]], rows_v.at[b], gsems[b])

    def wait_gather(b):
      pltpu.make_async_copy(h_hbm.at[dstb.at[0, 0]], rows_v.at[b],
                            gsems[b]).wait()

    def issue_scatter(i, b):
      pltpu.async_copy(rows_v.at[b], acc_s.at[srcb.at[i, 0]], ssems[b],
                       add=True)

    def wait_scatter(b):
      pltpu.make_async_copy(rows_v.at[b], acc_s.at[srcb.at[0, 0]],
                            ssems[b]).wait()

    def scale(i, b):
      def grp(g, c2):
        w16 = wb[i, 0, pl.ds(g * LANES, LANES)]
        for j in range(LANES):
          e = g * LANES + j
          wv = jnp.full((LANES,), w16[j], jnp.float32)
          for q in range(nq):
            sl = pl.ds(q * LANES, LANES)
            rows_v[b, e, sl] = rows_v[b, e, sl] * wv
        return c2

      lax.fori_loop(0, C // LANES, grp, 0)
      if C % LANES:
        off = C - LANES
        w16 = wb[i, 0, pl.ds(off, LANES)]
        for j in range(LANES - (C % LANES), LANES):
          e = off + j
          wv = jnp.full((LANES,), w16[j], jnp.float32)
          for q in range(nq):
            sl = pl.ds(q * LANES, LANES)
            rows_v[b, e, sl] = rows_v[b, e, sl] * wv

    def step(k, u):
      k = jnp.int32(k)
      b = u % NB
      b2 = (u + 2) % NB
      i2 = (u + 2) % NI
      i4 = (u + 4) % NI

      @pl.when(k >= 2)
      def _():
        wait_scatter(b2)

      @pl.when(k + 2 < n_chunks)
      def _():
        wait_idx(i2)
        issue_gather(i2, b2)

      @pl.when(k + 4 < n_chunks)
      def _():
        issue_idx(k + 4, i4)

      wait_gather(b)
      scale(u % NI, b)
      issue_scatter(u % NI, b)

    for j in range(4):
      issue_idx(j, j)
    wait_idx(0)
    issue_gather(0, 0)
    wait_idx(1)
    issue_gather(1, 1)

    n_main = (n_chunks // NI) * NI

    def outer(g, carry):
      for u in range(NI):
        step(g * NI + u, u)
      return carry

    lax.fori_loop(0, n_main // NI, outer, 0)

    for k in range(n_main, n_chunks):
      step(k, k % NI)

    for k in range(n_chunks - 2, n_chunks):
      wait_scatter(k % NB)
    plsc.subcore_barrier()

    def ocp(j, carry):
      c = j * NS + sid

      @pl.when(c < n_oc)
      def _():
        pltpu.sync_copy(acc_s.at[pl.ds(c * ZC, ZC)],
                        out_hbm.at[cid, pl.ds(c * ZC, ZC)])

      return carry

    lax.fori_loop(0, (n_oc + NS - 1) // NS, ocp, 0)

  return spmm


def _gathers_sc(V, N, Dm, B, L):
  T = B * L
  tpw = T // NW
  C = 128
  mpw = B // NW
  chunks = []
  off = 0
  while off < tpw:
    ln = min(C, tpw - off)
    chunks.append((off, ln))
    off += ln
  nch = len(chunks)
  NBG = 4

  @functools.partial(
      pl.kernel,
      out_type=(
          jax.ShapeDtypeStruct((T, Dm), jnp.float32),
          jax.ShapeDtypeStruct((B, Dm), jnp.float32),
          jax.ShapeDtypeStruct((B,), jnp.int32),
          jax.ShapeDtypeStruct((B,), jnp.float32),
      ),
      mesh=plsc.VectorSubcoreMesh(core_axis_name="c", subcore_axis_name="s"),
      scratch_types=[
          pltpu.VMEM((tpw,), jnp.int32),
          pltpu.VMEM((NBG, C, Dm), jnp.float32),
          pltpu.VMEM((mpw,), jnp.int32),
          pltpu.VMEM((mpw, Dm), jnp.float32),
          pltpu.VMEM((mpw,), jnp.int32),
          pltpu.VMEM((mpw,), jnp.float32),
          [pltpu.SemaphoreType.DMA] * NBG,
          [pltpu.SemaphoreType.DMA] * NBG,
          pltpu.SemaphoreType.DMA,
      ],
  )
  def g1(emb_hbm, v2f_hbm, ctx_hbm, miss_hbm,
         ctx_out, base_out, fg_out, mask_out,
         cidx_v, crows_v, midx_v, mrows_v, fg_v, mk_v, gsems, osems, sem):
    cid = lax.axis_index("c")
    sid = lax.axis_index("s")
    wid = cid * NS + sid
    base0 = wid * tpw

    pltpu.sync_copy(ctx_hbm.at[pl.ds(base0, tpw)], cidx_v)

    def issue_gather(k):
      o, ln = chunks[k]
      b = k % NBG
      pltpu.async_copy(emb_hbm.at[cidx_v.at[pl.ds(o, ln)]],
                       crows_v.at[b, pl.ds(0, ln)], gsems[b])

    def wait_gather(k):
      o, ln = chunks[k]
      b = k % NBG
      pltpu.make_async_copy(emb_hbm.at[cidx_v.at[pl.ds(o, ln)]],
                            crows_v.at[b, pl.ds(0, ln)], gsems[b]).wait()

    def issue_out(k):
      o, ln = chunks[k]
      b = k % NBG
      pltpu.async_copy(crows_v.at[b, pl.ds(0, ln)],
                       ctx_out.at[pl.ds(base0 + o, ln)], osems[b])

    def wait_out(k):
      o, ln = chunks[k]
      b = k % NBG
      pltpu.make_async_copy(crows_v.at[b, pl.ds(0, ln)],
                            ctx_out.at[pl.ds(base0 + o, ln)], osems[b]).wait()

    issue_gather(0)
    if nch > 1:
      issue_gather(1)
    for k in range(nch):
      wait_gather(k)
      issue_out(k)
      if k + 2 < nch:
        if k >= 2:
          wait_out(k - 2)
        issue_gather(k + 2)
    for k in range(max(0, nch - 4), nch):
      wait_out(k)

    mb = wid * mpw
    pltpu.sync_copy(miss_hbm.at[pl.ds(mb, mpw)], midx_v)
    pltpu.async_copy(emb_hbm.at[midx_v], mrows_v, sem).wait()
    pltpu.sync_copy(mrows_v, base_out.at[pl.ds(mb, mpw)])

    pltpu.async_copy(v2f_hbm.at[midx_v], fg_v, sem).wait()
    for g in range(mpw // LANES):
      fg = fg_v[pl.ds(g * LANES, LANES)]
      mk = jnp.where(fg >= 0, 1.0, 0.0).astype(jnp.float32)
      fgc = jnp.clip(fg, 0, N - 1)
      fg_v[pl.ds(g * LANES, LANES)] = fgc
      mk_v[pl.ds(g * LANES, LANES)] = mk
    pltpu.sync_copy(fg_v, fg_out.at[pl.ds(mb, mpw)])
    pltpu.sync_copy(mk_v, mask_out.at[pl.ds(mb, mpw)])

  return g1


def _gather_graph_sc(N, Dm, B):
  mpw = B // NW
  nq = Dm // LANES

  @functools.partial(
      pl.kernel,
      out_type=jax.ShapeDtypeStruct((B, Dm), jnp.float32),
      mesh=plsc.VectorSubcoreMesh(core_axis_name="c", subcore_axis_name="s"),
      scratch_types=[
          pltpu.VMEM((mpw,), jnp.int32),
          pltpu.VMEM((mpw,), jnp.float32),
          pltpu.VMEM((mpw, Dm), jnp.float32),
          pltpu.SemaphoreType.DMA,
      ],
  )
  def g2(ge_hbm, fg_hbm, mk_hbm, out_hbm, idx_v, mk_v, rows_v, sem):
    cid = lax.axis_index("c")
    sid = lax.axis_index("s")
    wid = cid * NS + sid
    mb = wid * mpw
    pltpu.sync_copy(fg_hbm.at[pl.ds(mb, mpw)], idx_v)
    pltpu.sync_copy(mk_hbm.at[pl.ds(mb, mpw)], mk_v)
    pltpu.async_copy(ge_hbm.at[idx_v], rows_v, sem).wait()

    def row(g, carry):
      mk16 = mk_v[pl.ds(g * LANES, LANES)]
      for j in range(LANES):
        e = g * LANES + j
        mv = jnp.full((LANES,), mk16[j], jnp.float32)
        for q in range(nq):
          sl = pl.ds(q * LANES, LANES)
          rows_v[e, sl] = rows_v[e, sl] * mv
      return carry

    lax.fori_loop(0, mpw // LANES, row, 0)
    pltpu.sync_copy(rows_v, out_hbm.at[pl.ds(mb, mpw)])

  return g2


def _layer_tc(N, Dm, bm):
  def body(p_ref, h_ref, W_ref, g_ref, b_ref, o_ref):
    s = p_ref[0] + p_ref[1]
    y = lax.dot_general(s, W_ref[...], (((1,), (1,)), ((), ())),
                        preferred_element_type=jnp.float32)
    y = jnp.maximum(y, 0.0)
    y = y * (g_ref[...] * BN_INV) + b_ref[...]
    o_ref[...] = h_ref[...] + y

  return pl.pallas_call(
      body,
      grid=(N // bm,),
      in_specs=[
          pl.BlockSpec((NC, bm, Dm), lambda i: (0, i, 0)),
          pl.BlockSpec((bm, Dm), lambda i: (i, 0)),
          pl.BlockSpec((Dm, Dm), lambda i: (0, 0)),
          pl.BlockSpec((1, Dm), lambda i: (0, 0)),
          pl.BlockSpec((1, Dm), lambda i: (0, 0)),
      ],
      out_specs=pl.BlockSpec((bm, Dm), lambda i: (i, 0)),
      out_shape=jax.ShapeDtypeStruct((N, Dm), jnp.float32),
  )


def _head_tc(B, L, Dm, bb):
  def body(c_ref, aw_ref, base_ref, gp_ref, fW_ref, fb_ref,
           p0W_ref, p0b_ref, p1W_ref, p1b_ref, o_ref):
    c = c_ref[...]
    lg = jnp.sum(c * aw_ref[...][0], axis=2, keepdims=True)
    m = jnp.max(lg, axis=0, keepdims=True)
    ex = jnp.exp(lg - m)
    a = ex / jnp.sum(ex, axis=0, keepdims=True)
    ctx_emb = jnp.sum(a * c, axis=0)
    x = jnp.concatenate([base_ref[...], gp_ref[...]], axis=1)
    miss = lax.dot_general(x, fW_ref[...], (((1,), (1,)), ((), ())),
                           preferred_element_type=jnp.float32) + fb_ref[...]
    q = jnp.concatenate([ctx_emb, miss], axis=1)
    hq = lax.dot_general(q, p0W_ref[...], (((1,), (1,)), ((), ())),
                         preferred_element_type=jnp.float32) + p0b_ref[...]
    hq = jnp.maximum(hq, 0.0)
    o_ref[...] = lax.dot_general(hq, p1W_ref[...], (((1,), (1,)), ((), ())),
                                 preferred_element_type=jnp.float32) + p1b_ref[...]

  full = lambda *s: pl.BlockSpec(s, lambda i: tuple(0 for _ in s))
  return pl.pallas_call(
      body,
      grid=(B // bb,),
      in_specs=[
          pl.BlockSpec((L, bb, Dm), lambda i: (0, i, 0)),
          full(1, Dm),
          pl.BlockSpec((bb, Dm), lambda i: (i, 0)),
          pl.BlockSpec((bb, Dm), lambda i: (i, 0)),
          full(Dm, 2 * Dm),
          full(1, Dm),
          full(2 * Dm, 2 * Dm),
          full(1, 2 * Dm),
          full(Dm, 2 * Dm),
          full(1, Dm),
      ],
      out_specs=pl.BlockSpec((bb, Dm), lambda i: (i, 0)),
      out_shape=jax.ShapeDtypeStruct((B, Dm), jnp.float32),
  )


def kernel(edge_index, edge_weight, ctx_ids, miss_ids, vocab_to_fg, embedding,
           fg_embedding, gc_W0, gc_W1, bn0_gamma, bn0_beta, bn1_gamma,
           bn1_beta, ctx_attn_W, ctx_attn_b, fusion_W, fusion_b, proj0_W,
           proj0_b, proj1_W, proj1_b):
  N, Dm = fg_embedding.shape
  E = edge_weight.shape[0]
  B, L = ctx_ids.shape
  V = embedding.shape[0]

  spmm = _spmm_sc(N, Dm, E)
  layer = _layer_tc(N, Dm, 1000)

  g0 = bn0_gamma.reshape(1, Dm)
  b0 = bn0_beta.reshape(1, Dm)
  g1w = bn1_gamma.reshape(1, Dm)
  b1w = bn1_beta.reshape(1, Dm)

  src_flat = edge_index[0]
  dst_flat = edge_index[1]
  h1 = layer(spmm(fg_embedding, dst_flat, src_flat, edge_weight),
             fg_embedding, gc_W0, g0, b0)
  graph_embs = layer(spmm(h1, dst_flat, src_flat, edge_weight),
                     h1, gc_W1, g1w, b1w)

  ctx_rows, base_emb, fg_clip, maskf = _gathers_sc(V, N, Dm, B, L)(
      embedding, vocab_to_fg, ctx_ids.T.reshape(B * L), miss_ids)
  gpart = _gather_graph_sc(N, Dm, B)(graph_embs, fg_clip, maskf)

  query = _head_tc(B, L, Dm, 128)(
      ctx_rows.reshape(L, B, Dm), ctx_attn_W, base_emb, gpart,
      fusion_W, fusion_b.reshape(1, Dm),
      proj0_W, proj0_b.reshape(1, 2 * Dm),
      proj1_W, proj1_b.reshape(1, Dm))

  return (query, graph_embs)


# speedup vs baseline: 9.6086x; 1.0446x over previous
import functools
import math

import jax
import jax.numpy as jnp
from jax import lax
from jax.experimental import pallas as pl
from jax.experimental.pallas import tpu as pltpu
from jax.experimental.pallas import tpu_sc as plsc

NC = 2
NS = 16
LANES = 16
NW = NC * NS
BN_INV = 1.0 / math.sqrt(1.0 + 1e-5)


def _spmm_sc(N, Dm, E):
  C = 40
  n_chunks = E // (NW * C)
  NB = 5
  NI = 10
  ZC = 200
  n_oc = N // ZC
  ZCz = C
  n_zc = N // ZCz
  nq = Dm // LANES

  @functools.partial(
      pl.kernel,
      out_type=jax.ShapeDtypeStruct((NC, N, Dm), jnp.float32),
      mesh=plsc.VectorSubcoreMesh(core_axis_name="c", subcore_axis_name="s"),
      scratch_types=[
          pltpu.VMEM((NI, 1, C), jnp.int32),
          pltpu.VMEM((NI, 1, C), jnp.int32),
          pltpu.VMEM((NI, 1, C), jnp.float32),
          pltpu.VMEM((NB, C, Dm), jnp.float32),
          pltpu.VMEM_SHARED((N, Dm), jnp.float32),
          [pltpu.SemaphoreType.DMA] * NB,
          [pltpu.SemaphoreType.DMA] * NB,
          [pltpu.SemaphoreType.DMA] * NI,
      ],
  )
  def spmm(h_hbm, dst_hbm, src_hbm, w_hbm, out_hbm,
           dstb, srcb, wb, rows_v, acc_s, gsems, ssems, isems):
    cid = lax.axis_index("c")
    sid = lax.axis_index("s")
    wid = cid * NS + sid

    zv = jnp.zeros((LANES,), jnp.float32)

    def zrow(r, carry):
      for q in range(nq):
        rows_v[0, r, pl.ds(q * LANES, LANES)] = zv
      return carry

    lax.fori_loop(0, C, zrow, 0)

    def zcp(j, carry):
      c = j * NS + sid

      @pl.when(c < n_zc)
      def _():
        pltpu.sync_copy(rows_v.at[0], acc_s.at[pl.ds(c * ZCz, ZCz)])

      return carry

    lax.fori_loop(0, (n_zc + NS - 1) // NS, zcp, 0)
    plsc.subcore_barrier()

    ebase = wid * (E // NW)

    def issue_idx(k, i):
      sl = pl.ds(ebase + k * C, C)
      pltpu.async_copy(dst_hbm.at[sl], dstb.at[i, 0], isems[i])
      pltpu.async_copy(src_hbm.at[sl], srcb.at[i, 0], isems[i])
      pltpu.async_copy(w_hbm.at[sl], wb.at[i, 0], isems[i])

    def wait_idx(i):
      sl = pl.ds(ebase, C)
      pltpu.make_async_copy(dst_hbm.at[sl], dstb.at[i, 0], isems[i]).wait()
      pltpu.make_async_copy(src_hbm.at[sl], srcb.at[i, 0], isems[i]).wait()
      pltpu.make_async_copy(w_hbm.at[sl], wb.at[i, 0], isems[i]).wait()

    def issue_gather(i, b):
      pltpu.async_copy(h_hbm.at[dstb.at[i, 0]], rows_v.at[b], gsems[b])

    def wait_gather(b):
      pltpu.make_async_copy(h_hbm.at[dstb.at[0, 0]], rows_v.at[b],
                            gsems[b]).wait()

    def issue_scatter(i, b):
      pltpu.async_copy(rows_v.at[b], acc_s.at[srcb.at[i, 0]], ssems[b],
                       add=True)

    def wait_scatter(b):
      pltpu.make_async_copy(rows_v.at[b], acc_s.at[srcb.at[0, 0]],
                            ssems[b]).wait()

    def scale(i, b):
      def grp(g, c2):
        w16 = wb[i, 0, pl.ds(g * LANES, LANES)]
        for j in range(LANES):
          e = g * LANES + j
          wv = jnp.full((LANES,), w16[j], jnp.float32)
          for q in range(nq):
            sl = pl.ds(q * LANES, LANES)
            rows_v[b, e, sl] = rows_v[b, e, sl] * wv
        return c2

      lax.fori_loop(0, C // LANES, grp, 0)
      if C % LANES:
        off = C - LANES
        w16 = wb[i, 0, pl.ds(off, LANES)]
        for j in range(LANES - (C % LANES), LANES):
          e = off + j
          wv = jnp.full((LANES,), w16[j], jnp.float32)
          for q in range(nq):
            sl = pl.ds(q * LANES, LANES)
            rows_v[b, e, sl] = rows_v[b, e, sl] * wv

    def step(k, u):
      k = jnp.int32(k)
      b = u % NB
      b3 = (u + 3) % NB
      i3 = (u + 3) % NI
      i5 = (u + 5) % NI

      @pl.when(k >= 2)
      def _():
        wait_scatter(b3)

      @pl.when(k + 3 < n_chunks)
      def _():
        wait_idx(i3)
        issue_gather(i3, b3)

      @pl.when(k + 5 < n_chunks)
      def _():
        issue_idx(k + 5, i5)

      wait_gather(b)
      scale(u % NI, b)
      issue_scatter(u % NI, b)

    for j in range(5):
      issue_idx(j, j)
    for j in range(3):
      wait_idx(j)
      issue_gather(j, j)

    n_main = (n_chunks // NI) * NI

    def outer(g, carry):
      for u in range(NI):
        step(g * NI + u, u)
      return carry

    lax.fori_loop(0, n_main // NI, outer, 0)

    for k in range(n_main, n_chunks):
      step(k, k % NI)

    for k in range(n_chunks - 2, n_chunks):
      wait_scatter(k % NB)
    plsc.subcore_barrier()

    def ocp(j, carry):
      c = j * NS + sid

      @pl.when(c < n_oc)
      def _():
        pltpu.sync_copy(acc_s.at[pl.ds(c * ZC, ZC)],
                        out_hbm.at[cid, pl.ds(c * ZC, ZC)])

      return carry

    lax.fori_loop(0, (n_oc + NS - 1) // NS, ocp, 0)

  return spmm


def _gathers_sc(V, N, Dm, B, L):
  T = B * L
  tpw = T // NW
  C = 128
  mpw = B // NW
  chunks = []
  off = 0
  while off < tpw:
    ln = min(C, tpw - off)
    chunks.append((off, ln))
    off += ln
  nch = len(chunks)
  NBG = 4

  @functools.partial(
      pl.kernel,
      out_type=(
          jax.ShapeDtypeStruct((T, Dm), jnp.float32),
          jax.ShapeDtypeStruct((B, Dm), jnp.float32),
          jax.ShapeDtypeStruct((B,), jnp.int32),
          jax.ShapeDtypeStruct((B,), jnp.float32),
      ),
      mesh=plsc.VectorSubcoreMesh(core_axis_name="c", subcore_axis_name="s"),
      scratch_types=[
          pltpu.VMEM((tpw,), jnp.int32),
          pltpu.VMEM((NBG, C, Dm), jnp.float32),
          pltpu.VMEM((mpw,), jnp.int32),
          pltpu.VMEM((mpw, Dm), jnp.float32),
          pltpu.VMEM((mpw,), jnp.int32),
          pltpu.VMEM((mpw,), jnp.float32),
          [pltpu.SemaphoreType.DMA] * NBG,
          [pltpu.SemaphoreType.DMA] * NBG,
          pltpu.SemaphoreType.DMA,
      ],
  )
  def g1(emb_hbm, v2f_hbm, ctx_hbm, miss_hbm,
         ctx_out, base_out, fg_out, mask_out,
         cidx_v, crows_v, midx_v, mrows_v, fg_v, mk_v, gsems, osems, sem):
    cid = lax.axis_index("c")
    sid = lax.axis_index("s")
    wid = cid * NS + sid
    base0 = wid * tpw

    pltpu.sync_copy(ctx_hbm.at[pl.ds(base0, tpw)], cidx_v)

    def issue_gather(k):
      o, ln = chunks[k]
      b = k % NBG
      pltpu.async_copy(emb_hbm.at[cidx_v.at[pl.ds(o, ln)]],
                       crows_v.at[b, pl.ds(0, ln)], gsems[b])

    def wait_gather(k):
      o, ln = chunks[k]
      b = k % NBG
      pltpu.make_async_copy(emb_hbm.at[cidx_v.at[pl.ds(o, ln)]],
                            crows_v.at[b, pl.ds(0, ln)], gsems[b]).wait()

    def issue_out(k):
      o, ln = chunks[k]
      b = k % NBG
      pltpu.async_copy(crows_v.at[b, pl.ds(0, ln)],
                       ctx_out.at[pl.ds(base0 + o, ln)], osems[b])

    def wait_out(k):
      o, ln = chunks[k]
      b = k % NBG
      pltpu.make_async_copy(crows_v.at[b, pl.ds(0, ln)],
                            ctx_out.at[pl.ds(base0 + o, ln)], osems[b]).wait()

    issue_gather(0)
    if nch > 1:
      issue_gather(1)
    for k in range(nch):
      wait_gather(k)
      issue_out(k)
      if k + 2 < nch:
        if k >= 2:
          wait_out(k - 2)
        issue_gather(k + 2)
    for k in range(max(0, nch - 4), nch):
      wait_out(k)

    mb = wid * mpw
    pltpu.sync_copy(miss_hbm.at[pl.ds(mb, mpw)], midx_v)
    pltpu.async_copy(emb_hbm.at[midx_v], mrows_v, sem).wait()
    pltpu.sync_copy(mrows_v, base_out.at[pl.ds(mb, mpw)])

    pltpu.async_copy(v2f_hbm.at[midx_v], fg_v, sem).wait()
    for g in range(mpw // LANES):
      fg = fg_v[pl.ds(g * LANES, LANES)]
      mk = jnp.where(fg >= 0, 1.0, 0.0).astype(jnp.float32)
      fgc = jnp.clip(fg, 0, N - 1)
      fg_v[pl.ds(g * LANES, LANES)] = fgc
      mk_v[pl.ds(g * LANES, LANES)] = mk
    pltpu.sync_copy(fg_v, fg_out.at[pl.ds(mb, mpw)])
    pltpu.sync_copy(mk_v, mask_out.at[pl.ds(mb, mpw)])

  return g1


def _gather_graph_sc(N, Dm, B):
  mpw = B // NW
  nq = Dm // LANES

  @functools.partial(
      pl.kernel,
      out_type=jax.ShapeDtypeStruct((B, Dm), jnp.float32),
      mesh=plsc.VectorSubcoreMesh(core_axis_name="c", subcore_axis_name="s"),
      scratch_types=[
          pltpu.VMEM((mpw,), jnp.int32),
          pltpu.VMEM((mpw,), jnp.float32),
          pltpu.VMEM((mpw, Dm), jnp.float32),
          pltpu.SemaphoreType.DMA,
      ],
  )
  def g2(ge_hbm, fg_hbm, mk_hbm, out_hbm, idx_v, mk_v, rows_v, sem):
    cid = lax.axis_index("c")
    sid = lax.axis_index("s")
    wid = cid * NS + sid
    mb = wid * mpw
    pltpu.sync_copy(fg_hbm.at[pl.ds(mb, mpw)], idx_v)
    pltpu.sync_copy(mk_hbm.at[pl.ds(mb, mpw)], mk_v)
    pltpu.async_copy(ge_hbm.at[idx_v], rows_v, sem).wait()

    def row(g, carry):
      mk16 = mk_v[pl.ds(g * LANES, LANES)]
      for j in range(LANES):
        e = g * LANES + j
        mv = jnp.full((LANES,), mk16[j], jnp.float32)
        for q in range(nq):
          sl = pl.ds(q * LANES, LANES)
          rows_v[e, sl] = rows_v[e, sl] * mv
      return carry

    lax.fori_loop(0, mpw // LANES, row, 0)
    pltpu.sync_copy(rows_v, out_hbm.at[pl.ds(mb, mpw)])

  return g2


def _layer_tc(N, Dm, bm):
  def body(p_ref, h_ref, W_ref, g_ref, b_ref, o_ref):
    s = p_ref[0] + p_ref[1]
    y = lax.dot_general(s, W_ref[...], (((1,), (1,)), ((), ())),
                        preferred_element_type=jnp.float32)
    y = jnp.maximum(y, 0.0)
    y = y * (g_ref[...] * BN_INV) + b_ref[...]
    o_ref[...] = h_ref[...] + y

  return pl.pallas_call(
      body,
      grid=(N // bm,),
      in_specs=[
          pl.BlockSpec((NC, bm, Dm), lambda i: (0, i, 0)),
          pl.BlockSpec((bm, Dm), lambda i: (i, 0)),
          pl.BlockSpec((Dm, Dm), lambda i: (0, 0)),
          pl.BlockSpec((1, Dm), lambda i: (0, 0)),
          pl.BlockSpec((1, Dm), lambda i: (0, 0)),
      ],
      out_specs=pl.BlockSpec((bm, Dm), lambda i: (i, 0)),
      out_shape=jax.ShapeDtypeStruct((N, Dm), jnp.float32),
  )


def _head_tc(B, L, Dm, bb):
  def body(c_ref, aw_ref, base_ref, gp_ref, fW_ref, fb_ref,
           p0W_ref, p0b_ref, p1W_ref, p1b_ref, o_ref):
    c = c_ref[...]
    lg = jnp.sum(c * aw_ref[...][0], axis=2, keepdims=True)
    m = jnp.max(lg, axis=0, keepdims=True)
    ex = jnp.exp(lg - m)
    a = ex / jnp.sum(ex, axis=0, keepdims=True)
    ctx_emb = jnp.sum(a * c, axis=0)
    x = jnp.concatenate([base_ref[...], gp_ref[...]], axis=1)
    miss = lax.dot_general(x, fW_ref[...], (((1,), (1,)), ((), ())),
                           preferred_element_type=jnp.float32) + fb_ref[...]
    q = jnp.concatenate([ctx_emb, miss], axis=1)
    hq = lax.dot_general(q, p0W_ref[...], (((1,), (1,)), ((), ())),
                         preferred_element_type=jnp.float32) + p0b_ref[...]
    hq = jnp.maximum(hq, 0.0)
    o_ref[...] = lax.dot_general(hq, p1W_ref[...], (((1,), (1,)), ((), ())),
                                 preferred_element_type=jnp.float32) + p1b_ref[...]

  full = lambda *s: pl.BlockSpec(s, lambda i: tuple(0 for _ in s))
  return pl.pallas_call(
      body,
      grid=(B // bb,),
      in_specs=[
          pl.BlockSpec((L, bb, Dm), lambda i: (0, i, 0)),
          full(1, Dm),
          pl.BlockSpec((bb, Dm), lambda i: (i, 0)),
          pl.BlockSpec((bb, Dm), lambda i: (i, 0)),
          full(Dm, 2 * Dm),
          full(1, Dm),
          full(2 * Dm, 2 * Dm),
          full(1, 2 * Dm),
          full(Dm, 2 * Dm),
          full(1, Dm),
      ],
      out_specs=pl.BlockSpec((bb, Dm), lambda i: (i, 0)),
      out_shape=jax.ShapeDtypeStruct((B, Dm), jnp.float32),
  )


def kernel(edge_index, edge_weight, ctx_ids, miss_ids, vocab_to_fg, embedding,
           fg_embedding, gc_W0, gc_W1, bn0_gamma, bn0_beta, bn1_gamma,
           bn1_beta, ctx_attn_W, ctx_attn_b, fusion_W, fusion_b, proj0_W,
           proj0_b, proj1_W, proj1_b):
  N, Dm = fg_embedding.shape
  E = edge_weight.shape[0]
  B, L = ctx_ids.shape
  V = embedding.shape[0]

  spmm = _spmm_sc(N, Dm, E)
  layer = _layer_tc(N, Dm, 1000)

  g0 = bn0_gamma.reshape(1, Dm)
  b0 = bn0_beta.reshape(1, Dm)
  g1w = bn1_gamma.reshape(1, Dm)
  b1w = bn1_beta.reshape(1, Dm)

  src_flat = edge_index[0]
  dst_flat = edge_index[1]
  h1 = layer(spmm(fg_embedding, dst_flat, src_flat, edge_weight),
             fg_embedding, gc_W0, g0, b0)
  graph_embs = layer(spmm(h1, dst_flat, src_flat, edge_weight),
                     h1, gc_W1, g1w, b1w)

  ctx_rows, base_emb, fg_clip, maskf = _gathers_sc(V, N, Dm, B, L)(
      embedding, vocab_to_fg, ctx_ids.T.reshape(B * L), miss_ids)
  gpart = _gather_graph_sc(N, Dm, B)(graph_embs, fg_clip, maskf)

  query = _head_tc(B, L, Dm, 128)(
      ctx_rows.reshape(L, B, Dm), ctx_attn_W, base_emb, gpart,
      fusion_W, fusion_b.reshape(1, Dm),
      proj0_W, proj0_b.reshape(1, 2 * Dm),
      proj1_W, proj1_b.reshape(1, Dm))

  return (query, graph_embs)
